# baseline (speedup 1.0000x reference)
import functools

import jax
import jax.numpy as jnp
from jax import lax
from jax.experimental import pallas as pl
from jax.experimental.pallas import tpu as pltpu
from jax.experimental.pallas import tpu_sc as plsc

N = 10000
E = 5000
NNZ = 320000
D = 128
H = 64
KEPT = (1, 2, 4, 8, 16)
NP = 10112
EP = 5120
NT = NP // 16
ET = EP // 16
SUB = 160
C = 400
PPT = NNZ // 16
NCH = PPT // C

NSUBS = [(0, 160), (160, 160), (320, 160), (480, 152)]
ESUBS = [(0, 160), (160, 160)]

_mesh = plsc.VectorSubcoreMesh(core_axis_name="c", subcore_axis_name="s")
_f32 = jnp.float32


def _zero_buf(buf, nrows):
    def body(r, _):
        for k in range(H // 16):
            buf[r, pl.ds(k * 16, 16)] = jnp.zeros((16,), _f32)
        return 0
    lax.fori_loop(0, nrows, body, 0)


def _scatter_phase(src, src_off, gidx, sidx, acc,
                   rows0, rows1, gib0, gib1, sib0, sib1, sem0, sem1):
    bufs = ((rows0, gib0, sib0, sem0), (rows1, gib1, sib1, sem1))

    def load_and_fire(ch, rows, gib, sib, sem):
        b = pl.multiple_of(ch * C, 8)
        pltpu.sync_copy(gidx.at[pl.ds(b, C)], gib)
        pltpu.sync_copy(sidx.at[pl.ds(b, C)], sib)
        def adj(k, _):
            gib[pl.ds(k * 16, 16)] = gib[pl.ds(k * 16, 16)] + src_off
            return 0
        lax.fori_loop(0, C // 16, adj, 0)
        pltpu.async_copy(src.at[gib], rows, sem)

    def drain_and_scatter(rows, sib, sem):
        pltpu.make_async_copy(src.at[pl.ds(0, C)], rows, sem).wait()
        pltpu.sync_copy(rows, acc.at[sib], add=True)

    load_and_fire(0, *bufs[0])

    def pipe(i, _):
        rA, gA, sA, mA = bufs[0]
        rB, gB, sB, mB = bufs[1]
        load_and_fire(2 * i + 1, rB, gB, sB, mB)
        drain_and_scatter(rA, sA, mA)
        @pl.when(i < NCH // 2 - 1)
        def _():
            load_and_fire(2 * i + 2, rA, gA, sA, mA)
        drain_and_scatter(rB, sB, mB)
        return 0
    lax.fori_loop(0, NCH // 2, pipe, 0)


def _deg_phase(idx, acc, ones, gib0, gib1):
    def chunk(i, _):
        b = pl.multiple_of(2 * i * C, 8)
        pltpu.sync_copy(idx.at[pl.ds(b, C)], gib0)
        b1 = pl.multiple_of((2 * i + 1) * C, 8)
        pltpu.sync_copy(idx.at[pl.ds(b1, C)], gib1)
        pltpu.sync_copy(ones, acc.at[gib0], add=True)
        pltpu.sync_copy(ones, acc.at[gib1], add=True)
        return 0
    lax.fori_loop(0, NCH // 2, chunk, 0)


def _inv_phase(acc, inv_out, buf, zbuf, o, subs):
    for (so, sn) in subs:
        oo = pl.multiple_of(o + so, 8)
        pltpu.sync_copy(acc.at[pl.ds(oo, sn)], buf.at[pl.ds(0, sn)])
        def body(r, _):
            for k in range(H // 16):
                d = buf[r, pl.ds(k * 16, 16)]
                buf[r, pl.ds(k * 16, 16)] = jnp.where(d > 0, 1.0 / d, 0.0)
            return 0
        lax.fori_loop(0, sn, body, 0)
        pltpu.sync_copy(buf.at[pl.ds(0, sn)], inv_out.at[pl.ds(oo, sn)])
        pltpu.sync_copy(zbuf.at[pl.ds(0, sn)], acc.at[pl.ds(oo, sn)])


def _combine_phase(acc, inv_hbm, state, state_off, o, subs,
                   buf, ibuf, zbuf, lvl_arr, core, slot):
    for (so, sn) in subs:
        oo = pl.multiple_of(o + so, 8)
        pltpu.sync_copy(acc.at[pl.ds(oo, sn)], buf.at[pl.ds(0, sn)])
        pltpu.sync_copy(buf.at[pl.ds(0, sn)],
                        lvl_arr.at[core, slot, pl.ds(oo, sn)])
        pltpu.sync_copy(inv_hbm.at[pl.ds(oo, sn)], ibuf.at[pl.ds(0, sn)])
        def body(r, _):
            for k in range(H // 16):
                buf[r, pl.ds(k * 16, 16)] = (buf[r, pl.ds(k * 16, 16)]
                                             * ibuf[r, pl.ds(k * 16, 16)])
            return 0
        lax.fori_loop(0, sn, body, 0)
        sso = pl.multiple_of(state_off + o + so, 8)
        pltpu.sync_copy(buf.at[pl.ds(0, sn)], state.at[pl.ds(sso, sn)])
        pltpu.sync_copy(zbuf.at[pl.ds(0, sn)], acc.at[pl.ds(oo, sn)])


@functools.partial(
    pl.kernel,
    out_type=(
        jax.ShapeDtypeStruct((2, 16, NP, H), _f32),
        jax.ShapeDtypeStruct((2, 16, EP, H), _f32),
        jax.ShapeDtypeStruct((2 * NP, H), _f32),
        jax.ShapeDtypeStruct((2 * EP, H), _f32),
        jax.ShapeDtypeStruct((NP, H), _f32),
        jax.ShapeDtypeStruct((EP, H), _f32),
    ),
    mesh=_mesh,
    scratch_types=[
        pltpu.VMEM((C, H), _f32),
        pltpu.VMEM((C, H), _f32),
        pltpu.VMEM((SUB, H), _f32),
        pltpu.VMEM((SUB, H), _f32),
        pltpu.VMEM((SUB, H), _f32),
        pltpu.VMEM((C,), jnp.int32),
        pltpu.VMEM((C,), jnp.int32),
        pltpu.VMEM((C,), jnp.int32),
        pltpu.VMEM((C,), jnp.int32),
        pltpu.VMEM_SHARED((NP, H), _f32),
        pltpu.SemaphoreType.DMA,
        pltpu.SemaphoreType.DMA,
    ],
    compiler_params=pltpu.CompilerParams(use_tc_tiling_on_sc=False),
)
def _diffusion(xh, iv, ie, lvl_n, lvl_e, state_n, state_e, inv_v, inv_e,
               rows0, rows1, buf, ibuf, zbuf, gib0, gib1, sib0, sib1, acc,
               sem0, sem1):
    c = lax.axis_index("c")
    s = lax.axis_index("s")
    no = pl.multiple_of(s * NT, 8)
    eo = pl.multiple_of(s * ET, 8)
    pb = pl.multiple_of(s * PPT, 8)
    ivs = iv.at[pl.ds(pb, PPT)]
    ies = ie.at[pl.ds(pb, PPT)]

    _zero_buf(zbuf, SUB)
    for (so, sn) in NSUBS:
        pltpu.sync_copy(zbuf.at[pl.ds(0, sn)],
                        acc.at[pl.ds(pl.multiple_of(no + so, 8), sn)])
    plsc.subcore_barrier()

    def fill(r, _):
        for k in range(H // 16):
            rows0[r, pl.ds(k * 16, 16)] = jnp.full((16,), 1.0, _f32)
        return 0
    lax.fori_loop(0, C, fill, 0)
    _deg_phase(ivs, acc, rows0, gib0, gib1)
    plsc.subcore_barrier()
    _inv_phase(acc, inv_v, buf, zbuf, no, NSUBS)
    plsc.subcore_barrier()
    _deg_phase(ies, acc, rows0, gib0, gib1)
    plsc.subcore_barrier()
    _inv_phase(acc, inv_e, buf, zbuf, eo, ESUBS)
    plsc.subcore_barrier()

    for (so, sn) in NSUBS:
        oo = pl.multiple_of(no + so, 8)
        pltpu.sync_copy(xh.at[c, pl.ds(oo, sn)], buf.at[pl.ds(0, sn)])
        pltpu.sync_copy(inv_v.at[pl.ds(oo, sn)], ibuf.at[pl.ds(0, sn)])
        def nrm(r, _):
            for k in range(H // 16):
                buf[r, pl.ds(k * 16, 16)] = (buf[r, pl.ds(k * 16, 16)]
                                             * ibuf[r, pl.ds(k * 16, 16)])
            return 0
        lax.fori_loop(0, sn, nrm, 0)
        sso = pl.multiple_of(c * NP + no + so, 8)
        pltpu.sync_copy(buf.at[pl.ds(0, sn)], state_n.at[pl.ds(sso, sn)])
    plsc.subcore_barrier()

    def round_body(t, _):
        _scatter_phase(state_n, c * NP, ivs, ies, acc,
                       rows0, rows1, gib0, gib1, sib0, sib1, sem0, sem1)
        plsc.subcore_barrier()
        _combine_phase(acc, inv_e, state_e, c * EP, eo, ESUBS, buf, ibuf,
                       zbuf, lvl_e, c, t)
        plsc.subcore_barrier()
        _scatter_phase(state_e, c * EP, ies, ivs, acc,
                       rows0, rows1, gib0, gib1, sib0, sib1, sem0, sem1)
        plsc.subcore_barrier()
        _combine_phase(acc, inv_v, state_n, c * NP, no, NSUBS, buf, ibuf,
                       zbuf, lvl_n, c, t)
        plsc.subcore_barrier()
        return 0
    lax.fori_loop(0, 16, round_body, 0)


def _wavelet_body(*refs):
    halves = refs[:12]
    out_ref = refs[12]
    for w in range(6):
        for h in range(2):
            cur = halves[h * 6 + w][...]
            coeff = cur - halves[h * 6 + w + 1][...] if w < 5 else cur
            pos = jnp.maximum(coeff, 0.0)
            neg = jnp.maximum(-coeff, 0.0)
            inter = jnp.stack([pos, neg], axis=-1).reshape(cur.shape[0], H * 2)
            base = w * D * 2 + h * H * 2
            out_ref[:, base:base + H * 2] = inter


def _wavelet(levels, rows):
    BLK = 200
    spec = pl.BlockSpec((BLK, H), lambda i: (i, 0))
    return pl.pallas_call(
        _wavelet_body,
        grid=(rows // BLK,),
        in_specs=[spec] * 12,
        out_specs=pl.BlockSpec((BLK, 6 * D * 2), lambda i: (i, 0)),
        out_shape=jax.ShapeDtypeStruct((rows, 6 * D * 2), _f32),
    )(*levels)


def kernel(X, Y, incidence_v, incidence_e, W):
    iv = incidence_v.astype(jnp.int32)
    ie = incidence_e.astype(jnp.int32)
    xh = jnp.stack([X[:, :H], X[:, H:]])
    xh = jnp.pad(xh, ((0, 0), (0, NP - N), (0, 0)))

    lvl_n, lvl_e, _, _, _, _ = _diffusion(xh, iv, ie)

    def half_levels(base, lvls, rows):
        out = []
        for h in range(2):
            out.append(base[:, h * H:(h + 1) * H])
            for t in KEPT:
                out.append(lvls[h, t - 1, :rows, :])
        return out

    s_nodes = _wavelet(half_levels(X, lvl_n, N), N)
    s_edges = _wavelet(half_levels(Y, lvl_e, E), E)
    return (s_nodes, s_edges)


# speedup vs baseline: 1.0838x; 1.0838x over previous
import functools

import jax
import jax.numpy as jnp
from jax import lax
from jax.experimental import pallas as pl
from jax.experimental.pallas import tpu as pltpu
from jax.experimental.pallas import tpu_sc as plsc

N = 10000
E = 5000
NNZ = 320000
D = 128
H = 64
KEPT = (1, 2, 4, 8, 16)
NP = 10112
EP = 5120
NT = NP // 16
ET = EP // 16
SUB = 160
C = 400
PPT = NNZ // 16
NCH = PPT // C

NSUBS = [(0, 160), (160, 160), (320, 160), (480, 152)]
ESUBS = [(0, 160), (160, 160)]

_mesh = plsc.VectorSubcoreMesh(core_axis_name="c", subcore_axis_name="s")
_f32 = jnp.float32


def _zero_buf(buf, nrows):
    def body(r, _):
        for k in range(H // 16):
            buf[r, pl.ds(k * 16, 16)] = jnp.zeros((16,), _f32)
        return 0
    lax.fori_loop(0, nrows, body, 0)


def _scatter_phase(src, src_off, gidx, sidx, acc,
                   rows0, rows1, gib0, gib1, sib0, sib1, sem0, sem1):
    bufs = ((rows0, gib0, sib0, sem0), (rows1, gib1, sib1, sem1))

    def load_and_fire(ch, rows, gib, sib, sem):
        b = pl.multiple_of(ch * C, 8)
        pltpu.sync_copy(gidx.at[pl.ds(b, C)], gib)
        pltpu.sync_copy(sidx.at[pl.ds(b, C)], sib)
        def adj(k, _):
            gib[pl.ds(k * 16, 16)] = gib[pl.ds(k * 16, 16)] + src_off
            return 0
        lax.fori_loop(0, C // 16, adj, 0)
        pltpu.async_copy(src.at[gib], rows, sem)

    def drain_and_scatter(rows, sib, sem):
        pltpu.make_async_copy(src.at[pl.ds(0, C)], rows, sem).wait()
        pltpu.sync_copy(rows, acc.at[sib], add=True)

    load_and_fire(0, *bufs[0])

    def pipe(i, _):
        rA, gA, sA, mA = bufs[0]
        rB, gB, sB, mB = bufs[1]
        load_and_fire(2 * i + 1, rB, gB, sB, mB)
        drain_and_scatter(rA, sA, mA)
        @pl.when(i < NCH // 2 - 1)
        def _():
            load_and_fire(2 * i + 2, rA, gA, sA, mA)
        drain_and_scatter(rB, sB, mB)
        return 0
    lax.fori_loop(0, NCH // 2, pipe, 0)


def _deg_phase(idx, acc, ones, gib0, gib1):
    def chunk(i, _):
        b = pl.multiple_of(2 * i * C, 8)
        pltpu.sync_copy(idx.at[pl.ds(b, C)], gib0)
        b1 = pl.multiple_of((2 * i + 1) * C, 8)
        pltpu.sync_copy(idx.at[pl.ds(b1, C)], gib1)
        pltpu.sync_copy(ones, acc.at[gib0], add=True)
        pltpu.sync_copy(ones, acc.at[gib1], add=True)
        return 0
    lax.fori_loop(0, NCH // 2, chunk, 0)


def _inv_phase(acc, inv_out, buf, zbuf, o, subs):
    for (so, sn) in subs:
        oo = pl.multiple_of(o + so, 8)
        pltpu.sync_copy(acc.at[pl.ds(oo, sn)], buf.at[pl.ds(0, sn)])
        def body(r, _):
            for k in range(H // 16):
                d = buf[r, pl.ds(k * 16, 16)]
                buf[r, pl.ds(k * 16, 16)] = jnp.where(d > 0, 1.0 / d, 0.0)
            return 0
        lax.fori_loop(0, sn, body, 0)
        pltpu.sync_copy(buf.at[pl.ds(0, sn)], inv_out.at[pl.ds(oo, sn)])
        pltpu.sync_copy(zbuf.at[pl.ds(0, sn)], acc.at[pl.ds(oo, sn)])


def _combine_phase(acc, inv_hbm, state, state_off, o, subs,
                   buf, ibuf, zbuf, lvl_arr, core, slot):
    for (so, sn) in subs:
        oo = pl.multiple_of(o + so, 8)
        pltpu.sync_copy(acc.at[pl.ds(oo, sn)], buf.at[pl.ds(0, sn)])
        pltpu.sync_copy(buf.at[pl.ds(0, sn)],
                        lvl_arr.at[core, slot, pl.ds(oo, sn)])
        pltpu.sync_copy(inv_hbm.at[pl.ds(oo, sn)], ibuf.at[pl.ds(0, sn)])
        def body(r, _):
            for k in range(H // 16):
                buf[r, pl.ds(k * 16, 16)] = (buf[r, pl.ds(k * 16, 16)]
                                             * ibuf[r, pl.ds(k * 16, 16)])
            return 0
        lax.fori_loop(0, sn, body, 0)
        sso = pl.multiple_of(state_off + o + so, 8)
        pltpu.sync_copy(buf.at[pl.ds(0, sn)], state.at[pl.ds(sso, sn)])
        pltpu.sync_copy(zbuf.at[pl.ds(0, sn)], acc.at[pl.ds(oo, sn)])


@functools.partial(
    pl.kernel,
    out_type=(
        jax.ShapeDtypeStruct((2, 16, NP, H), _f32),
        jax.ShapeDtypeStruct((2, 16, EP, H), _f32),
        jax.ShapeDtypeStruct((2 * NP, H), _f32),
        jax.ShapeDtypeStruct((2 * EP, H), _f32),
        jax.ShapeDtypeStruct((NP, H), _f32),
        jax.ShapeDtypeStruct((EP, H), _f32),
    ),
    mesh=_mesh,
    scratch_types=[
        pltpu.VMEM((C, H), _f32),
        pltpu.VMEM((C, H), _f32),
        pltpu.VMEM((SUB, H), _f32),
        pltpu.VMEM((SUB, H), _f32),
        pltpu.VMEM((SUB, H), _f32),
        pltpu.VMEM((C,), jnp.int32),
        pltpu.VMEM((C,), jnp.int32),
        pltpu.VMEM((C,), jnp.int32),
        pltpu.VMEM((C,), jnp.int32),
        pltpu.VMEM_SHARED((NP, H), _f32),
        pltpu.SemaphoreType.DMA,
        pltpu.SemaphoreType.DMA,
    ],
    compiler_params=pltpu.CompilerParams(use_tc_tiling_on_sc=False),
)
def _diffusion(xh, iv, ie, lvl_n, lvl_e, state_n, state_e, inv_v, inv_e,
               rows0, rows1, buf, ibuf, zbuf, gib0, gib1, sib0, sib1, acc,
               sem0, sem1):
    c = lax.axis_index("c")
    s = lax.axis_index("s")
    no = pl.multiple_of(s * NT, 8)
    eo = pl.multiple_of(s * ET, 8)
    pb = pl.multiple_of(s * PPT, 8)
    ivs = iv.at[pl.ds(pb, PPT)]
    ies = ie.at[pl.ds(pb, PPT)]

    _zero_buf(zbuf, SUB)
    for (so, sn) in NSUBS:
        pltpu.sync_copy(zbuf.at[pl.ds(0, sn)],
                        acc.at[pl.ds(pl.multiple_of(no + so, 8), sn)])
    plsc.subcore_barrier()

    def fill(r, _):
        for k in range(H // 16):
            rows0[r, pl.ds(k * 16, 16)] = jnp.full((16,), 1.0, _f32)
        return 0
    lax.fori_loop(0, C, fill, 0)
    _deg_phase(ivs, acc, rows0, gib0, gib1)
    plsc.subcore_barrier()
    _inv_phase(acc, inv_v, buf, zbuf, no, NSUBS)
    plsc.subcore_barrier()
    _deg_phase(ies, acc, rows0, gib0, gib1)
    plsc.subcore_barrier()
    _inv_phase(acc, inv_e, buf, zbuf, eo, ESUBS)
    plsc.subcore_barrier()

    for (so, sn) in NSUBS:
        oo = pl.multiple_of(no + so, 8)
        pltpu.sync_copy(xh.at[c, pl.ds(oo, sn)], buf.at[pl.ds(0, sn)])
        pltpu.sync_copy(inv_v.at[pl.ds(oo, sn)], ibuf.at[pl.ds(0, sn)])
        def nrm(r, _):
            for k in range(H // 16):
                buf[r, pl.ds(k * 16, 16)] = (buf[r, pl.ds(k * 16, 16)]
                                             * ibuf[r, pl.ds(k * 16, 16)])
            return 0
        lax.fori_loop(0, sn, nrm, 0)
        sso = pl.multiple_of(c * NP + no + so, 8)
        pltpu.sync_copy(buf.at[pl.ds(0, sn)], state_n.at[pl.ds(sso, sn)])
    plsc.subcore_barrier()

    def round_body(t, _):
        _scatter_phase(state_n, c * NP, ivs, ies, acc,
                       rows0, rows1, gib0, gib1, sib0, sib1, sem0, sem1)
        plsc.subcore_barrier()
        _combine_phase(acc, inv_e, state_e, c * EP, eo, ESUBS, buf, ibuf,
                       zbuf, lvl_e, c, t)
        plsc.subcore_barrier()
        _scatter_phase(state_e, c * EP, ies, ivs, acc,
                       rows0, rows1, gib0, gib1, sib0, sib1, sem0, sem1)
        plsc.subcore_barrier()
        _combine_phase(acc, inv_v, state_n, c * NP, no, NSUBS, buf, ibuf,
                       zbuf, lvl_n, c, t)
        plsc.subcore_barrier()
        return 0
    lax.fori_loop(0, 16, round_body, 0)


def _wavelet_body(*refs):
    halves = refs[:12]
    out_ref = refs[12]
    for w in range(6):
        for h in range(2):
            cur = halves[h * 6 + w][...]
            coeff = cur - halves[h * 6 + w + 1][...] if w < 5 else cur
            pos = jnp.maximum(coeff, 0.0)
            neg = jnp.maximum(-coeff, 0.0)
            inter = jnp.stack([pos, neg], axis=-1).reshape(cur.shape[0], H * 2)
            base = w * D * 2 + h * H * 2
            out_ref[:, base:base + H * 2] = inter


def _wavelet(levels, rows):
    BLK = 40
    spec = pl.BlockSpec((BLK, H), lambda i: (i, 0))
    return pl.pallas_call(
        _wavelet_body,
        grid=(rows // BLK,),
        in_specs=[spec] * 12,
        out_specs=pl.BlockSpec((BLK, 6 * D * 2), lambda i: (i, 0)),
        out_shape=jax.ShapeDtypeStruct((rows, 6 * D * 2), _f32),
    )(*levels)


def kernel(X, Y, incidence_v, incidence_e, W):
    iv = incidence_v.astype(jnp.int32)
    ie = incidence_e.astype(jnp.int32)
    xh = jnp.stack([X[:, :H], X[:, H:]])
    xh = jnp.pad(xh, ((0, 0), (0, NP - N), (0, 0)))

    lvl_n, lvl_e, _, _, _, _ = _diffusion(xh, iv, ie)

    def half_levels(base, lvls, rows):
        out = []
        for h in range(2):
            out.append(base[:, h * H:(h + 1) * H])
            for t in KEPT:
                out.append(lvls[h, t - 1, :rows, :])
        return out

    s_nodes = _wavelet(half_levels(X, lvl_n, N), N)
    s_edges = _wavelet(half_levels(Y, lvl_e, E), E)
    return (s_nodes, s_edges)


# speedup vs baseline: 1.0960x; 1.0112x over previous
import functools

import jax
import jax.numpy as jnp
from jax import lax
from jax.experimental import pallas as pl
from jax.experimental.pallas import tpu as pltpu
from jax.experimental.pallas import tpu_sc as plsc

N = 10000
E = 5000
NNZ = 320000
D = 128
H = 64
KEPT = (1, 2, 4, 8, 16)
NP = 10112
EP = 5120
NT = NP // 16
ET = EP // 16
SUB = 160
C = 400
PPT = NNZ // 16
NCH = PPT // C

NSUBS = [(0, 160), (160, 160), (320, 160), (480, 152)]
ESUBS = [(0, 160), (160, 160)]

_mesh = plsc.VectorSubcoreMesh(core_axis_name="c", subcore_axis_name="s")
_f32 = jnp.float32


def _zero_buf(buf, nrows):
    def body(r, _):
        for k in range(H // 16):
            buf[r, pl.ds(k * 16, 16)] = jnp.zeros((16,), _f32)
        return 0
    lax.fori_loop(0, nrows, body, 0)


def _scatter_phase(src, src_off, gidx, sidx, acc,
                   rows0, rows1, gib0, gib1, sib0, sib1, sem0, sem1):
    bufs = ((rows0, gib0, sib0, sem0), (rows1, gib1, sib1, sem1))

    def load_and_fire(ch, rows, gib, sib, sem):
        b = pl.multiple_of(ch * C, 8)
        pltpu.sync_copy(gidx.at[pl.ds(b, C)], gib)
        pltpu.sync_copy(sidx.at[pl.ds(b, C)], sib)
        def adj(k, _):
            gib[pl.ds(k * 16, 16)] = gib[pl.ds(k * 16, 16)] + src_off
            return 0
        lax.fori_loop(0, C // 16, adj, 0)
        pltpu.async_copy(src.at[gib], rows, sem)

    def drain_and_scatter(rows, sib, sem):
        pltpu.make_async_copy(src.at[pl.ds(0, C)], rows, sem).wait()
        pltpu.sync_copy(rows, acc.at[sib], add=True)

    load_and_fire(0, *bufs[0])

    def pipe(i, _):
        rA, gA, sA, mA = bufs[0]
        rB, gB, sB, mB = bufs[1]
        load_and_fire(2 * i + 1, rB, gB, sB, mB)
        drain_and_scatter(rA, sA, mA)
        @pl.when(i < NCH // 2 - 1)
        def _():
            load_and_fire(2 * i + 2, rA, gA, sA, mA)
        drain_and_scatter(rB, sB, mB)
        return 0
    lax.fori_loop(0, NCH // 2, pipe, 0)


def _deg_phase(idx, acc, ones, gib0, gib1):
    def chunk(i, _):
        b = pl.multiple_of(2 * i * C, 8)
        pltpu.sync_copy(idx.at[pl.ds(b, C)], gib0)
        b1 = pl.multiple_of((2 * i + 1) * C, 8)
        pltpu.sync_copy(idx.at[pl.ds(b1, C)], gib1)
        pltpu.sync_copy(ones, acc.at[gib0], add=True)
        pltpu.sync_copy(ones, acc.at[gib1], add=True)
        return 0
    lax.fori_loop(0, NCH // 2, chunk, 0)


def _inv_phase(acc, inv_out, buf, zbuf, o, subs):
    for (so, sn) in subs:
        oo = pl.multiple_of(o + so, 8)
        pltpu.sync_copy(acc.at[pl.ds(oo, sn)], buf.at[pl.ds(0, sn)])
        def body(r, _):
            for k in range(H // 16):
                d = buf[r, pl.ds(k * 16, 16)]
                buf[r, pl.ds(k * 16, 16)] = jnp.where(d > 0, 1.0 / d, 0.0)
            return 0
        lax.fori_loop(0, sn, body, 0)
        pltpu.sync_copy(buf.at[pl.ds(0, sn)], inv_out.at[pl.ds(oo, sn)])
        pltpu.sync_copy(zbuf.at[pl.ds(0, sn)], acc.at[pl.ds(oo, sn)])


def _combine_phase(acc, inv_hbm, state, state_off, o, subs,
                   buf, ibuf, zbuf, lvl_arr, core, slot, sem0, sem1):
    for (so, sn) in subs:
        oo = pl.multiple_of(o + so, 8)
        d_acc = pltpu.async_copy(acc.at[pl.ds(oo, sn)],
                                 buf.at[pl.ds(0, sn)], sem0)
        d_inv = pltpu.async_copy(inv_hbm.at[pl.ds(oo, sn)],
                                 ibuf.at[pl.ds(0, sn)], sem1)
        d_acc.wait()
        d_inv.wait()
        def body(r, _):
            for k in range(H // 16):
                ibuf[r, pl.ds(k * 16, 16)] = (buf[r, pl.ds(k * 16, 16)]
                                              * ibuf[r, pl.ds(k * 16, 16)])
            return 0
        lax.fori_loop(0, sn, body, 0)
        sso = pl.multiple_of(state_off + o + so, 8)
        d_lvl = pltpu.async_copy(buf.at[pl.ds(0, sn)],
                                 lvl_arr.at[core, slot, pl.ds(oo, sn)], sem0)
        d_st = pltpu.async_copy(ibuf.at[pl.ds(0, sn)],
                                state.at[pl.ds(sso, sn)], sem1)
        pltpu.sync_copy(zbuf.at[pl.ds(0, sn)], acc.at[pl.ds(oo, sn)])
        d_lvl.wait()
        d_st.wait()


@functools.partial(
    pl.kernel,
    out_type=(
        jax.ShapeDtypeStruct((2, 16, NP, H), _f32),
        jax.ShapeDtypeStruct((2, 16, EP, H), _f32),
        jax.ShapeDtypeStruct((2 * NP, H), _f32),
        jax.ShapeDtypeStruct((2 * EP, H), _f32),
        jax.ShapeDtypeStruct((NP, H), _f32),
        jax.ShapeDtypeStruct((EP, H), _f32),
    ),
    mesh=_mesh,
    scratch_types=[
        pltpu.VMEM((C, H), _f32),
        pltpu.VMEM((C, H), _f32),
        pltpu.VMEM((SUB, H), _f32),
        pltpu.VMEM((SUB, H), _f32),
        pltpu.VMEM((SUB, H), _f32),
        pltpu.VMEM((C,), jnp.int32),
        pltpu.VMEM((C,), jnp.int32),
        pltpu.VMEM((C,), jnp.int32),
        pltpu.VMEM((C,), jnp.int32),
        pltpu.VMEM_SHARED((NP, H), _f32),
        pltpu.SemaphoreType.DMA,
        pltpu.SemaphoreType.DMA,
    ],
    compiler_params=pltpu.CompilerParams(use_tc_tiling_on_sc=False),
)
def _diffusion(xh, iv, ie, lvl_n, lvl_e, state_n, state_e, inv_v, inv_e,
               rows0, rows1, buf, ibuf, zbuf, gib0, gib1, sib0, sib1, acc,
               sem0, sem1):
    c = lax.axis_index("c")
    s = lax.axis_index("s")
    no = pl.multiple_of(s * NT, 8)
    eo = pl.multiple_of(s * ET, 8)
    pb = pl.multiple_of(s * PPT, 8)
    ivs = iv.at[pl.ds(pb, PPT)]
    ies = ie.at[pl.ds(pb, PPT)]

    _zero_buf(zbuf, SUB)
    for (so, sn) in NSUBS:
        pltpu.sync_copy(zbuf.at[pl.ds(0, sn)],
                        acc.at[pl.ds(pl.multiple_of(no + so, 8), sn)])
    plsc.subcore_barrier()

    def fill(r, _):
        for k in range(H // 16):
            rows0[r, pl.ds(k * 16, 16)] = jnp.full((16,), 1.0, _f32)
        return 0
    lax.fori_loop(0, C, fill, 0)
    _deg_phase(ivs, acc, rows0, gib0, gib1)
    plsc.subcore_barrier()
    _inv_phase(acc, inv_v, buf, zbuf, no, NSUBS)
    plsc.subcore_barrier()
    _deg_phase(ies, acc, rows0, gib0, gib1)
    plsc.subcore_barrier()
    _inv_phase(acc, inv_e, buf, zbuf, eo, ESUBS)
    plsc.subcore_barrier()

    for (so, sn) in NSUBS:
        oo = pl.multiple_of(no + so, 8)
        pltpu.sync_copy(xh.at[c, pl.ds(oo, sn)], buf.at[pl.ds(0, sn)])
        pltpu.sync_copy(inv_v.at[pl.ds(oo, sn)], ibuf.at[pl.ds(0, sn)])
        def nrm(r, _):
            for k in range(H // 16):
                buf[r, pl.ds(k * 16, 16)] = (buf[r, pl.ds(k * 16, 16)]
                                             * ibuf[r, pl.ds(k * 16, 16)])
            return 0
        lax.fori_loop(0, sn, nrm, 0)
        sso = pl.multiple_of(c * NP + no + so, 8)
        pltpu.sync_copy(buf.at[pl.ds(0, sn)], state_n.at[pl.ds(sso, sn)])
    plsc.subcore_barrier()

    def round_body(t, _):
        _scatter_phase(state_n, c * NP, ivs, ies, acc,
                       rows0, rows1, gib0, gib1, sib0, sib1, sem0, sem1)
        plsc.subcore_barrier()
        _combine_phase(acc, inv_e, state_e, c * EP, eo, ESUBS, buf, ibuf,
                       zbuf, lvl_e, c, t, sem0, sem1)
        plsc.subcore_barrier()
        _scatter_phase(state_e, c * EP, ies, ivs, acc,
                       rows0, rows1, gib0, gib1, sib0, sib1, sem0, sem1)
        plsc.subcore_barrier()
        _combine_phase(acc, inv_v, state_n, c * NP, no, NSUBS, buf, ibuf,
                       zbuf, lvl_n, c, t, sem0, sem1)
        plsc.subcore_barrier()
        return 0
    lax.fori_loop(0, 16, round_body, 0)


def _wavelet_body(*refs):
    halves = refs[:12]
    out_ref = refs[12]
    for w in range(6):
        for h in range(2):
            cur = halves[h * 6 + w][...]
            coeff = cur - halves[h * 6 + w + 1][...] if w < 5 else cur
            pos = jnp.maximum(coeff, 0.0)
            neg = jnp.maximum(-coeff, 0.0)
            inter = jnp.stack([pos, neg], axis=-1).reshape(cur.shape[0], H * 2)
            base = w * D * 2 + h * H * 2
            out_ref[:, base:base + H * 2] = inter


def _wavelet(levels, rows):
    BLK = 40
    spec = pl.BlockSpec((BLK, H), lambda i: (i, 0))
    return pl.pallas_call(
        _wavelet_body,
        grid=(rows // BLK,),
        in_specs=[spec] * 12,
        out_specs=pl.BlockSpec((BLK, 6 * D * 2), lambda i: (i, 0)),
        out_shape=jax.ShapeDtypeStruct((rows, 6 * D * 2), _f32),
    )(*levels)


def kernel(X, Y, incidence_v, incidence_e, W):
    iv = incidence_v.astype(jnp.int32)
    ie = incidence_e.astype(jnp.int32)
    xh = jnp.stack([X[:, :H], X[:, H:]])
    xh = jnp.pad(xh, ((0, 0), (0, NP - N), (0, 0)))

    lvl_n, lvl_e, _, _, _, _ = _diffusion(xh, iv, ie)

    def half_levels(base, lvls, rows):
        out = []
        for h in range(2):
            out.append(base[:, h * H:(h + 1) * H])
            for t in KEPT:
                out.append(lvls[h, t - 1, :rows, :])
        return out

    s_nodes = _wavelet(half_levels(X, lvl_n, N), N)
    s_edges = _wavelet(half_levels(Y, lvl_e, E), E)
    return (s_nodes, s_edges)


# speedup vs baseline: 1.1039x; 1.0072x over previous
import functools

import jax
import jax.numpy as jnp
from jax import lax
from jax.experimental import pallas as pl
from jax.experimental.pallas import tpu as pltpu
from jax.experimental.pallas import tpu_sc as plsc

N = 10000
E = 5000
NNZ = 320000
D = 128
H = 64
KEPT = (1, 2, 4, 8, 16)
NP = 10112
EP = 5120
NT = NP // 16
ET = EP // 16
SUB = 160
C = 400
PPT = NNZ // 16
NCH = PPT // C

NSUBS = [(0, 160), (160, 160), (320, 160), (480, 152)]
ESUBS = [(0, 160), (160, 160)]

_mesh = plsc.VectorSubcoreMesh(core_axis_name="c", subcore_axis_name="s")
_f32 = jnp.float32


def _zero_buf(buf, nrows):
    def body(r, _):
        for k in range(H // 16):
            buf[r, pl.ds(k * 16, 16)] = jnp.zeros((16,), _f32)
        return 0
    lax.fori_loop(0, nrows, body, 0)


def _scatter_phase(src, src_off, gidx, sidx, acc,
                   rows0, rows1, gib0, gib1, sib0, sib1, sem0, sem1):
    bufs = ((rows0, gib0, sib0, sem0), (rows1, gib1, sib1, sem1))

    def load_and_fire(ch, rows, gib, sib, sem):
        b = pl.multiple_of(ch * C, 8)
        pltpu.sync_copy(gidx.at[pl.ds(b, C)], gib)
        pltpu.sync_copy(sidx.at[pl.ds(b, C)], sib)
        def adj(k, _):
            gib[pl.ds(k * 16, 16)] = gib[pl.ds(k * 16, 16)] + src_off
            return 0
        lax.fori_loop(0, C // 16, adj, 0)
        pltpu.async_copy(src.at[gib], rows, sem)

    def drain_and_scatter(rows, sib, sem):
        pltpu.make_async_copy(src.at[pl.ds(0, C)], rows, sem).wait()
        pltpu.sync_copy(rows, acc.at[sib], add=True)

    load_and_fire(0, *bufs[0])

    def pipe(i, _):
        rA, gA, sA, mA = bufs[0]
        rB, gB, sB, mB = bufs[1]
        load_and_fire(2 * i + 1, rB, gB, sB, mB)
        drain_and_scatter(rA, sA, mA)
        @pl.when(i < NCH // 2 - 1)
        def _():
            load_and_fire(2 * i + 2, rA, gA, sA, mA)
        drain_and_scatter(rB, sB, mB)
        return 0
    lax.fori_loop(0, NCH // 2, pipe, 0)


def _deg_phase(idx, acc, ones, gib0, gib1):
    def chunk(i, _):
        b = pl.multiple_of(2 * i * C, 8)
        pltpu.sync_copy(idx.at[pl.ds(b, C)], gib0)
        b1 = pl.multiple_of((2 * i + 1) * C, 8)
        pltpu.sync_copy(idx.at[pl.ds(b1, C)], gib1)
        pltpu.sync_copy(ones, acc.at[gib0], add=True)
        pltpu.sync_copy(ones, acc.at[gib1], add=True)
        return 0
    lax.fori_loop(0, NCH // 2, chunk, 0)


def _inv_phase(acc, inv_out, buf, zbuf, o, subs):
    for (so, sn) in subs:
        oo = pl.multiple_of(o + so, 8)
        pltpu.sync_copy(acc.at[pl.ds(oo, sn)], buf.at[pl.ds(0, sn)])
        def body(r, _):
            for k in range(H // 16):
                d = buf[r, pl.ds(k * 16, 16)]
                buf[r, pl.ds(k * 16, 16)] = jnp.where(d > 0, 1.0 / d, 0.0)
            return 0
        lax.fori_loop(0, sn, body, 0)
        pltpu.sync_copy(buf.at[pl.ds(0, sn)], inv_out.at[pl.ds(oo, sn)])
        pltpu.sync_copy(zbuf.at[pl.ds(0, sn)], acc.at[pl.ds(oo, sn)])


def _combine_phase(acc, inv_hbm, state, state_off, o, subs,
                   buf, ibuf, zbuf, lvl_arr, core, slot, sem0, sem1):
    for (so, sn) in subs:
        oo = pl.multiple_of(o + so, 8)
        d_acc = pltpu.async_copy(acc.at[pl.ds(oo, sn)],
                                 buf.at[pl.ds(0, sn)], sem0)
        d_inv = pltpu.async_copy(inv_hbm.at[pl.ds(oo, sn)],
                                 ibuf.at[pl.ds(0, sn)], sem1)
        d_acc.wait()
        d_inv.wait()
        def body(r, _):
            for k in range(H // 16):
                ibuf[r, pl.ds(k * 16, 16)] = (buf[r, pl.ds(k * 16, 16)]
                                              * ibuf[r, pl.ds(k * 16, 16)])
            return 0
        lax.fori_loop(0, sn, body, 0)
        sso = pl.multiple_of(state_off + o + so, 8)
        d_lvl = pltpu.async_copy(buf.at[pl.ds(0, sn)],
                                 lvl_arr.at[core, slot, pl.ds(oo, sn)], sem0)
        d_st = pltpu.async_copy(ibuf.at[pl.ds(0, sn)],
                                state.at[pl.ds(sso, sn)], sem1)
        pltpu.sync_copy(zbuf.at[pl.ds(0, sn)], acc.at[pl.ds(oo, sn)])
        d_lvl.wait()
        d_st.wait()


@functools.partial(
    pl.kernel,
    out_type=(
        jax.ShapeDtypeStruct((2, 16, NP, H), _f32),
        jax.ShapeDtypeStruct((2, 16, EP, H), _f32),
        jax.ShapeDtypeStruct((2 * NP, H), _f32),
        jax.ShapeDtypeStruct((2 * EP, H), _f32),
        jax.ShapeDtypeStruct((NP, H), _f32),
        jax.ShapeDtypeStruct((EP, H), _f32),
    ),
    mesh=_mesh,
    scratch_types=[
        pltpu.VMEM((C, H), _f32),
        pltpu.VMEM((C, H), _f32),
        pltpu.VMEM((SUB, H), _f32),
        pltpu.VMEM((SUB, H), _f32),
        pltpu.VMEM((SUB, H), _f32),
        pltpu.VMEM((C,), jnp.int32),
        pltpu.VMEM((C,), jnp.int32),
        pltpu.VMEM((C,), jnp.int32),
        pltpu.VMEM((C,), jnp.int32),
        pltpu.VMEM_SHARED((NP, H), _f32),
        pltpu.SemaphoreType.DMA,
        pltpu.SemaphoreType.DMA,
    ],
    compiler_params=pltpu.CompilerParams(use_tc_tiling_on_sc=False),
)
def _diffusion(xh, iv, ie, lvl_n, lvl_e, state_n, state_e, inv_v, inv_e,
               rows0, rows1, buf, ibuf, zbuf, gib0, gib1, sib0, sib1, acc,
               sem0, sem1):
    c = lax.axis_index("c")
    s = lax.axis_index("s")
    no = pl.multiple_of(s * NT, 8)
    eo = pl.multiple_of(s * ET, 8)
    pb = pl.multiple_of(s * PPT, 8)
    ivs = iv.at[pl.ds(pb, PPT)]
    ies = ie.at[pl.ds(pb, PPT)]

    _zero_buf(zbuf, SUB)
    for (so, sn) in NSUBS:
        pltpu.sync_copy(zbuf.at[pl.ds(0, sn)],
                        acc.at[pl.ds(pl.multiple_of(no + so, 8), sn)])
    plsc.subcore_barrier()

    def fill(r, _):
        for k in range(H // 16):
            rows0[r, pl.ds(k * 16, 16)] = jnp.full((16,), 1.0, _f32)
        return 0
    lax.fori_loop(0, C, fill, 0)
    _deg_phase(ivs, acc, rows0, gib0, gib1)
    plsc.subcore_barrier()
    _inv_phase(acc, inv_v, buf, zbuf, no, NSUBS)
    plsc.subcore_barrier()
    _deg_phase(ies, acc, rows0, gib0, gib1)
    plsc.subcore_barrier()
    _inv_phase(acc, inv_e, buf, zbuf, eo, ESUBS)
    plsc.subcore_barrier()

    for (so, sn) in NSUBS:
        oo = pl.multiple_of(no + so, 8)
        pltpu.sync_copy(xh.at[c, pl.ds(oo, sn)], buf.at[pl.ds(0, sn)])
        pltpu.sync_copy(inv_v.at[pl.ds(oo, sn)], ibuf.at[pl.ds(0, sn)])
        def nrm(r, _):
            for k in range(H // 16):
                buf[r, pl.ds(k * 16, 16)] = (buf[r, pl.ds(k * 16, 16)]
                                             * ibuf[r, pl.ds(k * 16, 16)])
            return 0
        lax.fori_loop(0, sn, nrm, 0)
        sso = pl.multiple_of(c * NP + no + so, 8)
        pltpu.sync_copy(buf.at[pl.ds(0, sn)], state_n.at[pl.ds(sso, sn)])
    plsc.subcore_barrier()

    def round_body(t, _):
        _scatter_phase(state_n, c * NP, ivs, ies, acc,
                       rows0, rows1, gib0, gib1, sib0, sib1, sem0, sem1)
        plsc.subcore_barrier()
        _combine_phase(acc, inv_e, state_e, c * EP, eo, ESUBS, buf, ibuf,
                       zbuf, lvl_e, c, t, sem0, sem1)
        plsc.subcore_barrier()
        _scatter_phase(state_e, c * EP, ies, ivs, acc,
                       rows0, rows1, gib0, gib1, sib0, sib1, sem0, sem1)
        plsc.subcore_barrier()
        _combine_phase(acc, inv_v, state_n, c * NP, no, NSUBS, buf, ibuf,
                       zbuf, lvl_n, c, t, sem0, sem1)
        plsc.subcore_barrier()
        return 0
    lax.fori_loop(0, 16, round_body, 0)


def _wavelet_body(*refs):
    halves = refs[:12]
    out_ref = refs[12]
    for w in range(6):
        for h in range(2):
            cur = halves[h * 6 + w][...]
            coeff = cur - halves[h * 6 + w + 1][...] if w < 5 else cur
            pos = jnp.maximum(coeff, 0.0)
            neg = jnp.maximum(-coeff, 0.0)
            inter = jnp.stack([pos, neg], axis=-1).reshape(cur.shape[0], H * 2)
            base = w * D * 2 + h * H * 2
            out_ref[:, base:base + H * 2] = inter


def _wavelet(levels, rows, BLK):
    spec = pl.BlockSpec((BLK, H), lambda i: (i, 0))
    return pl.pallas_call(
        _wavelet_body,
        grid=(rows // BLK,),
        in_specs=[spec] * 12,
        out_specs=pl.BlockSpec((BLK, 6 * D * 2), lambda i: (i, 0)),
        out_shape=jax.ShapeDtypeStruct((rows, 6 * D * 2), _f32),
    )(*levels)


def kernel(X, Y, incidence_v, incidence_e, W):
    iv = incidence_v.astype(jnp.int32)
    ie = incidence_e.astype(jnp.int32)
    xh = jnp.stack([X[:, :H], X[:, H:]])
    xh = jnp.pad(xh, ((0, 0), (0, NP - N), (0, 0)))

    lvl_n, lvl_e, _, _, _, _ = _diffusion(xh, iv, ie)

    def half_levels(base, lvls, rows):
        out = []
        for h in range(2):
            out.append(base[:, h * H:(h + 1) * H])
            for t in KEPT:
                out.append(lvls[h, t - 1, :rows, :])
        return out

    s_nodes = _wavelet(half_levels(X, lvl_n, N), N, 80)
    s_edges = _wavelet(half_levels(Y, lvl_e, E), E, 40)
    return (s_nodes, s_edges)
